# Initial kernel scaffold; baseline (speedup 1.0000x reference)
import jax
import jax.numpy as jnp
from jax.experimental import pallas as pl


def kernel(x, edge_index, W1, b1, W2, b2, W3, b3, Wv, bv, Wp, bp):
    raise NotImplementedError("write your pallas kernel here")


# speedup vs baseline: 43.8603x; 43.8603x over previous
import functools

import jax
import jax.numpy as jnp
from jax import lax
from jax.experimental import pallas as pl
from jax.experimental.pallas import tpu as pltpu
from jax.experimental.pallas import tpu_sc as plsc

def _mesh():
    return plsc.VectorSubcoreMesh(core_axis_name="c", subcore_axis_name="s")


_NB = 8
_CH = 128
_NWORKERS = 32


def _make_prop(NP, C, rw):
    nbody = rw // _NB
    rpt = NP // 16
    one_d = C == 1
    tshape = (NP,) if one_d else (NP, C)
    oshape = (2, NP) if one_d else (2, NP, C)
    bshape = (_NB, _CH) if one_d else (_NB, _CH, C)

    @functools.partial(
        pl.kernel,
        out_type=jax.ShapeDtypeStruct(oshape, jnp.float32),
        mesh=_mesh(),
        scratch_types=[
            pltpu.VMEM((_NB, _CH), jnp.int32),
            pltpu.VMEM((_NB, _CH), jnp.int32),
            pltpu.VMEM(bshape, jnp.float32),
            pltpu.VMEM_SHARED(tshape, jnp.float32),
            pltpu.SemaphoreType.DMA,
            pltpu.SemaphoreType.DMA,
        ],
        compiler_params=pltpu.CompilerParams(use_tc_tiling_on_sc=False),
    )
    def prop(z_hbm, src_hbm, dst_hbm, zeros_hbm, out_hbm,
             src_v, dst_v, buf_v, acc, gsem, ssem):
        cid = lax.axis_index("c")
        sid = lax.axis_index("s")
        w = sid * 2 + cid
        pltpu.sync_copy(zeros_hbm.at[pl.ds(sid * rpt, rpt)],
                        acc.at[pl.ds(sid * rpt, rpt)])
        plsc.subcore_barrier()
        base = w * rw

        def body(j, carry):
            r0 = base + j * _NB
            pltpu.sync_copy(src_hbm.at[pl.ds(r0, _NB)], src_v)
            pltpu.sync_copy(dst_hbm.at[pl.ds(r0, _NB)], dst_v)
            g = [pltpu.async_copy(z_hbm.at[src_v.at[k]], buf_v.at[k], gsem)
                 for k in range(_NB)]
            s = []
            for k in range(_NB):
                g[k].wait()
                s.append(pltpu.async_copy(buf_v.at[k], acc.at[dst_v.at[k]],
                                          ssem, add=True))
            for h in s:
                h.wait()
            return carry

        lax.fori_loop(0, nbody, body, 0)
        plsc.subcore_barrier()
        pltpu.sync_copy(acc.at[pl.ds(sid * rpt, rpt)],
                        out_hbm.at[cid, pl.ds(sid * rpt, rpt)])

    return prop


def _make_deg(NP, rw):
    nbody = rw // _NB
    rpt = NP // 16

    @functools.partial(
        pl.kernel,
        out_type=jax.ShapeDtypeStruct((2, NP), jnp.float32),
        mesh=_mesh(),
        scratch_types=[
            pltpu.VMEM((_NB, _CH), jnp.int32),
            pltpu.VMEM((_CH,), jnp.float32),
            pltpu.VMEM_SHARED((NP,), jnp.float32),
            pltpu.SemaphoreType.DMA,
        ],
        compiler_params=pltpu.CompilerParams(use_tc_tiling_on_sc=False),
    )
    def deg(dst_hbm, ones_hbm, zeros_hbm, out_hbm, dst_v, ones_v, acc, ssem):
        cid = lax.axis_index("c")
        sid = lax.axis_index("s")
        w = sid * 2 + cid
        pltpu.sync_copy(zeros_hbm.at[pl.ds(sid * rpt, rpt)],
                        acc.at[pl.ds(sid * rpt, rpt)])
        pltpu.sync_copy(ones_hbm, ones_v)
        plsc.subcore_barrier()
        base = w * rw

        def body(j, carry):
            r0 = base + j * _NB
            pltpu.sync_copy(dst_hbm.at[pl.ds(r0, _NB)], dst_v)
            s = [pltpu.async_copy(ones_v, acc.at[dst_v.at[k]], ssem, add=True)
                 for k in range(_NB)]
            for h in s:
                h.wait()
            return carry

        lax.fori_loop(0, nbody, body, 0)
        plsc.subcore_barrier()
        pltpu.sync_copy(acc.at[pl.ds(sid * rpt, rpt)],
                        out_hbm.at[cid, pl.ds(sid * rpt, rpt)])

    return deg


def _tc0(degp, xp, R, grid):
    def body(degp_ref, x_ref, dis_ref, z1_ref):
        deg = degp_ref[0] + degp_ref[1] + 1.0
        dis = lax.rsqrt(deg)
        dis_ref[...] = dis
        z1_ref[...] = dis * x_ref[...]

    return pl.pallas_call(
        body,
        grid=(grid,),
        in_specs=[
            pl.BlockSpec((2, R, 1), lambda i: (0, i, 0)),
            pl.BlockSpec((R, 1), lambda i: (i, 0)),
        ],
        out_specs=[pl.BlockSpec((R, 1), lambda i: (i, 0)),
                   pl.BlockSpec((R, 1), lambda i: (i, 0))],
        out_shape=[jax.ShapeDtypeStruct((R * grid, 1), jnp.float32),
                   jax.ShapeDtypeStruct((R * grid, 1), jnp.float32)],
    )(degp, xp)


def _tc1(s1c, z1c, disc, W1, b1, W2, R, grid):
    def body(s1_ref, z1_ref, dis_ref, w1_ref, b1_ref, w2_ref, za_ref, zb_ref):
        dc = dis_ref[...]
        pc = dc * (s1_ref[0] + s1_ref[1] + z1_ref[...])
        h1 = jnp.maximum(pc * w1_ref[...] + b1_ref[...], 0.0)
        z2 = dc * jnp.dot(h1, w2_ref[...], preferred_element_type=jnp.float32)
        za_ref[...] = z2[:, :16]
        zb_ref[...] = z2[:, 16:]

    return pl.pallas_call(
        body,
        grid=(grid,),
        in_specs=[
            pl.BlockSpec((2, R, 1), lambda i: (0, i, 0)),
            pl.BlockSpec((R, 1), lambda i: (i, 0)),
            pl.BlockSpec((R, 1), lambda i: (i, 0)),
            pl.BlockSpec((1, 64), lambda i: (0, 0)),
            pl.BlockSpec((1, 64), lambda i: (0, 0)),
            pl.BlockSpec((64, 32), lambda i: (0, 0)),
        ],
        out_specs=[pl.BlockSpec((R, 16), lambda i: (i, 0)),
                   pl.BlockSpec((R, 16), lambda i: (i, 0))],
        out_shape=[jax.ShapeDtypeStruct((R * grid, 16), jnp.float32),
                   jax.ShapeDtypeStruct((R * grid, 16), jnp.float32)],
    )(s1c, z1c, disc, W1, b1.reshape(1, 64), W2)


def _tc2(s2a, s2b, z2a, z2b, disc, b2, W3, R, grid):
    def body(sa_ref, sb_ref, za_ref, zb_ref, dis_ref, b2_ref, w3_ref, z3_ref):
        dc = dis_ref[...]
        ha = dc * (sa_ref[0] + sa_ref[1] + za_ref[...])
        hb = dc * (sb_ref[0] + sb_ref[1] + zb_ref[...])
        h2 = jnp.maximum(jnp.concatenate([ha, hb], axis=1) + b2_ref[...], 0.0)
        z3_ref[...] = dc * jnp.dot(h2, w3_ref[...],
                                   preferred_element_type=jnp.float32)

    return pl.pallas_call(
        body,
        grid=(grid,),
        in_specs=[
            pl.BlockSpec((2, R, 16), lambda i: (0, i, 0)),
            pl.BlockSpec((2, R, 16), lambda i: (0, i, 0)),
            pl.BlockSpec((R, 16), lambda i: (i, 0)),
            pl.BlockSpec((R, 16), lambda i: (i, 0)),
            pl.BlockSpec((R, 1), lambda i: (i, 0)),
            pl.BlockSpec((1, 32), lambda i: (0, 0)),
            pl.BlockSpec((32, 16), lambda i: (0, 0)),
        ],
        out_specs=pl.BlockSpec((R, 16), lambda i: (i, 0)),
        out_shape=jax.ShapeDtypeStruct((R * grid, 16), jnp.float32),
    )(s2a, s2b, z2a, z2b, disc, b2.reshape(1, 32), W3)


def _tc3(s3p, z3, disc, b3, Wp, n, R, grid):
    def body(s3_ref, z3_ref, dis_ref, b3_ref, wp_ref, zp_ref, hsum_ref):
        i = pl.program_id(0)
        dc = dis_ref[...]
        h3 = jnp.maximum(dc * (s3_ref[0] + s3_ref[1] + z3_ref[...])
                         + b3_ref[...], 0.0)
        zp_ref[...] = dc * jnp.dot(h3, wp_ref[...],
                                   preferred_element_type=jnp.float32)
        gidx = i * R + lax.broadcasted_iota(jnp.int32, (R, 1), 0)
        h3m = jnp.where(gidx < n, h3, 0.0)
        colsum = jnp.sum(h3m, axis=0, keepdims=True)

        @pl.when(i == 0)
        def _():
            hsum_ref[...] = jnp.zeros_like(hsum_ref)

        hsum_ref[...] += colsum

    return pl.pallas_call(
        body,
        grid=(grid,),
        in_specs=[
            pl.BlockSpec((2, R, 16), lambda i: (0, i, 0)),
            pl.BlockSpec((R, 16), lambda i: (i, 0)),
            pl.BlockSpec((R, 1), lambda i: (i, 0)),
            pl.BlockSpec((1, 16), lambda i: (0, 0)),
            pl.BlockSpec((16, 1), lambda i: (0, 0)),
        ],
        out_specs=[pl.BlockSpec((R, 1), lambda i: (i, 0)),
                   pl.BlockSpec((1, 16), lambda i: (0, 0))],
        out_shape=[jax.ShapeDtypeStruct((R * grid, 1), jnp.float32),
                   jax.ShapeDtypeStruct((1, 16), jnp.float32)],
    )(s3p, z3, disc, b3.reshape(1, 16), Wp)


def _tc4(spc, zpc, disc, bp, hsum, Wv, bv, n, R, grid):
    def body(sp_ref, zp_ref, dis_ref, bp_ref, hsum_ref, wv_ref, bv_ref,
             pol_ref, val_ref):
        i = pl.program_id(0)
        pol_ref[...] = (dis_ref[...] * (sp_ref[0] + sp_ref[1] + zp_ref[...])
                        + bp_ref[0, 0])

        @pl.when(i == 0)
        def _():
            m = hsum_ref[...] * (1.0 / n)
            v = jnp.dot(m, wv_ref[...], preferred_element_type=jnp.float32)
            val_ref[...] = jax.nn.sigmoid(v + bv_ref[...])

    return pl.pallas_call(
        body,
        grid=(grid,),
        in_specs=[
            pl.BlockSpec((2, R, 1), lambda i: (0, i, 0)),
            pl.BlockSpec((R, 1), lambda i: (i, 0)),
            pl.BlockSpec((R, 1), lambda i: (i, 0)),
            pl.BlockSpec((1, 1), lambda i: (0, 0)),
            pl.BlockSpec((1, 16), lambda i: (0, 0)),
            pl.BlockSpec((16, 1), lambda i: (0, 0)),
            pl.BlockSpec((1, 1), lambda i: (0, 0)),
        ],
        out_specs=[pl.BlockSpec((R, 1), lambda i: (i, 0)),
                   pl.BlockSpec((1, 1), lambda i: (0, 0))],
        out_shape=[jax.ShapeDtypeStruct((R * grid, 1), jnp.float32),
                   jax.ShapeDtypeStruct((1, 1), jnp.float32)],
    )(spc, zpc, disc, bp.reshape(1, 1), hsum, Wv, bv.reshape(1, 1))


def kernel(x, edge_index, W1, b1, W2, b2, W3, b3, Wv, bv, Wp, bp):
    n = x.shape[0]
    e = edge_index.shape[1]
    grid = 25
    R = 2048
    NP = -(-(n + 1) // (grid * R)) * (grid * R)
    rw = -(-e // (_NWORKERS * _CH * _NB)) * _NB
    EPR = _NWORKERS * rw

    src = edge_index[0].astype(jnp.int32)
    dst = edge_index[1].astype(jnp.int32)
    pad = EPR * _CH - e
    padv = jnp.full((pad,), n, jnp.int32)
    srcp = jnp.concatenate([src, padv]).reshape(EPR, _CH)
    dstp = jnp.concatenate([dst, padv]).reshape(EPR, _CH)

    xp = jnp.pad(x, ((0, NP - n), (0, 0)))
    zeros1 = jnp.zeros((NP,), jnp.float32)
    zeros16 = jnp.zeros((NP, 16), jnp.float32)
    ones = jnp.ones((_CH,), jnp.float32)

    prop1 = _make_prop(NP, 1, rw)
    prop16 = _make_prop(NP, 16, rw)
    degk = _make_deg(NP, rw)

    degp = degk(dstp, ones, zeros1)
    disc, z1c = _tc0(degp.reshape(2, NP, 1), xp, R, grid)

    s1 = prop1(z1c.reshape(NP), srcp, dstp, zeros1)
    z2a, z2b = _tc1(s1.reshape(2, NP, 1), z1c, disc, W1, b1, W2, R, grid)

    s2a = prop16(z2a, srcp, dstp, zeros16)
    s2b = prop16(z2b, srcp, dstp, zeros16)
    z3 = _tc2(s2a, s2b, z2a, z2b, disc, b2, W3, R, grid)

    s3 = prop16(z3, srcp, dstp, zeros16)
    zpc, hsum = _tc3(s3, z3, disc, b3, Wp, n, R, grid)

    sp = prop1(zpc.reshape(NP), srcp, dstp, zeros1)
    polc, val = _tc4(sp.reshape(2, NP, 1), zpc, disc, bp, hsum, Wv, bv,
                     n, R, grid)

    policy = polc[:n, 0]
    value = val.reshape(1)
    return (value, policy)
